# baseline (speedup 1.0000x reference)
import functools
import jax
import jax.numpy as jnp
from jax import lax
from jax.experimental import pallas as pl
from jax.experimental.pallas import tpu as pltpu
from jax.experimental.pallas import tpu_sc as plsc

_N, _L, _C = 16, 4096, 64
_BN = 4
_NC, _NS = 2, 16
_TH = _L // 2


def _sc_count_body(strip_hbm, cnt_out, buf, cres, sem):
    c = lax.axis_index("c")
    s = lax.axis_index("s")
    wid = s * _NC + c
    n = wid // 2
    t0 = (wid % 2) * _TH
    pltpu.async_copy(strip_hbm.at[n, pl.ds(t0, _TH)], buf, sem).wait()

    def body(i, carry):
        bits = buf[pl.ds(i * 16, 16)]
        nan = (bits & jnp.int32(0x7FFFFFFF)) > jnp.int32(0x7F800000)
        return carry + jnp.where(nan, 0.0, 1.0)

    ca = lax.fori_loop(0, _TH // 16, body, jnp.zeros((16,), jnp.float32))
    cres[...] = ca
    pltpu.sync_copy(cres, cnt_out.at[pl.ds(wid * 16, 16)])


@functools.cache
def _sc_count():
    return pl.kernel(
        _sc_count_body,
        mesh=plsc.VectorSubcoreMesh(core_axis_name="c", subcore_axis_name="s"),
        out_type=jax.ShapeDtypeStruct((_NC * _NS * 16,), jnp.float32),
        scratch_types=[
            pltpu.VMEM((_TH,), jnp.int32),
            pltpu.VMEM((16,), jnp.float32),
            pltpu.SemaphoreType.DMA,
        ],
    )


def _body(p_ref, l_ref, c_ref, out_ref, acc_ref):
    step = pl.program_id(0)

    @pl.when(step == 0)
    def _init():
        acc_ref[0] = 0.0

    l = l_ref[...]
    p = p_ref[...]
    nan = jnp.isnan(l)
    d = jnp.where(nan, 0.0, p - l)
    acc_ref[0] += jnp.sum(d * d)

    @pl.when(step == pl.num_programs(0) - 1)
    def _fin():
        out_ref[0] = acc_ref[0] / (float(_C) * jnp.sum(c_ref[...]))


def kernel(preds, labels):
    pt = preds.swapaxes(1, 2)
    lt = labels.swapaxes(1, 2)
    strip = jax.lax.bitcast_convert_type(labels[:, :, 0], jnp.int32)
    cnts = _sc_count()(strip)
    out = pl.pallas_call(
        _body,
        grid=(_N // _BN,),
        in_specs=[
            pl.BlockSpec((_BN, _C, _L), lambda i: (i, 0, 0)),
            pl.BlockSpec((_BN, _C, _L), lambda i: (i, 0, 0)),
            pl.BlockSpec((_NC * _NS, 16), lambda i: (0, 0)),
        ],
        out_specs=pl.BlockSpec(memory_space=pltpu.SMEM),
        out_shape=jax.ShapeDtypeStruct((1,), jnp.float32),
        scratch_shapes=[pltpu.SMEM((2,), jnp.float32)],
    )(pt, lt, cnts.reshape(_NC * _NS, 16))
    return out[0]
